# baseline (speedup 1.0000x reference)
import jax
import jax.numpy as jnp
from jax.experimental import pallas as pl
from jax.experimental.pallas import tpu as pltpu

_BM = 256

_Q2 = 0.1673855234853893
_Q1 = -0.46665652800123064
_Q0 = 0.9999999333816031


def _stripe_kernel(scale_ref, bias_ref, img_ref, txt_ref, out_ref, txt_bf):
    i = pl.program_id(0)

    @pl.when(i == 0)
    def _():
        txt_bf[...] = txt_ref[...].astype(jnp.bfloat16)

    scale = scale_ref[0]
    bias = bias_ref[0]
    img_blk = (img_ref[...] * scale).astype(jnp.bfloat16)
    x = jax.lax.dot_general(
        img_blk, txt_bf[...],
        (((1,), (1,)), ((), ())),
        preferred_element_type=jnp.float32,
    )
    x = x + bias
    t = jnp.abs(x)
    u = jnp.exp(-t)
    q = (_Q2 * u + _Q1) * u + _Q0
    body = jnp.maximum(x, 0.0) + u * q
    txt_slice = txt_ref[pl.ds(i * _BM, _BM), :]
    trace = scale * jnp.sum(img_ref[...] * txt_slice) + _BM * bias
    out_ref[i] = jnp.sum(body) - trace


def kernel(image_features, text_features, logit_scale, logit_bias):
    n, d = image_features.shape
    scale = jnp.reshape(logit_scale.astype(jnp.float32), (1,))
    bias = jnp.reshape(logit_bias.astype(jnp.float32), (1,))
    grid = n // _BM
    partials = pl.pallas_call(
        _stripe_kernel,
        grid=(grid,),
        in_specs=[
            pl.BlockSpec(memory_space=pltpu.SMEM),
            pl.BlockSpec(memory_space=pltpu.SMEM),
            pl.BlockSpec((_BM, d), lambda i: (i, 0)),
            pl.BlockSpec(text_features.shape, lambda i: (0, 0)),
        ],
        out_specs=pl.BlockSpec(memory_space=pltpu.SMEM),
        out_shape=jax.ShapeDtypeStruct((grid,), jnp.float32),
        scratch_shapes=[pltpu.VMEM((n, d), jnp.bfloat16)],
        compiler_params=pltpu.CompilerParams(
            dimension_semantics=("arbitrary",),
        ),
    )(scale, bias, image_features, text_features)
    return jnp.sum(partials) / n


# speedup vs baseline: 1.1263x; 1.1263x over previous
import jax
import jax.numpy as jnp
from jax.experimental import pallas as pl
from jax.experimental.pallas import tpu as pltpu

_BM = 1024

_Q2 = 0.1673855234853893
_Q1 = -0.46665652800123064
_Q0 = 0.9999999333816031


def _stripe_kernel(scale_ref, bias_ref, img_ref, txt_ref, out_ref, txt_bf):
    i = pl.program_id(0)

    @pl.when(i == 0)
    def _():
        txt_bf[...] = txt_ref[...].astype(jnp.bfloat16)

    scale = scale_ref[0]
    bias = bias_ref[0]
    img_blk = (img_ref[...] * scale).astype(jnp.bfloat16)
    x = jax.lax.dot_general(
        img_blk, txt_bf[...],
        (((1,), (1,)), ((), ())),
        preferred_element_type=jnp.float32,
    )
    x = x + bias
    t = jnp.abs(x)
    u = jnp.exp(-t)
    q = (_Q2 * u + _Q1) * u + _Q0
    body = jnp.maximum(x, 0.0) + u * q
    txt_slice = txt_ref[pl.ds(i * _BM, _BM), :]
    trace = scale * jnp.sum(img_ref[...] * txt_slice) + _BM * bias
    out_ref[i] = jnp.sum(body) - trace


def kernel(image_features, text_features, logit_scale, logit_bias):
    n, d = image_features.shape
    scale = jnp.reshape(logit_scale.astype(jnp.float32), (1,))
    bias = jnp.reshape(logit_bias.astype(jnp.float32), (1,))
    grid = n // _BM
    partials = pl.pallas_call(
        _stripe_kernel,
        grid=(grid,),
        in_specs=[
            pl.BlockSpec(memory_space=pltpu.SMEM),
            pl.BlockSpec(memory_space=pltpu.SMEM),
            pl.BlockSpec((_BM, d), lambda i: (i, 0)),
            pl.BlockSpec(text_features.shape, lambda i: (0, 0)),
        ],
        out_specs=pl.BlockSpec(memory_space=pltpu.SMEM),
        out_shape=jax.ShapeDtypeStruct((grid,), jnp.float32),
        scratch_shapes=[pltpu.VMEM((n, d), jnp.bfloat16)],
        compiler_params=pltpu.CompilerParams(
            dimension_semantics=("arbitrary",),
        ),
    )(scale, bias, image_features, text_features)
    return jnp.sum(partials) / n


# speedup vs baseline: 1.2651x; 1.1232x over previous
import jax
import jax.numpy as jnp
from jax.experimental import pallas as pl
from jax.experimental.pallas import tpu as pltpu

_BM = 1024
_BN = 1024

_Q2 = 0.1673855234853893
_Q1 = -0.46665652800123064
_Q0 = 0.9999999333816031


def _stripe_kernel(scale_ref, bias_ref, img_ref, txt_ref, out_ref, txt_bf):
    i = pl.program_id(0)

    @pl.when(i == 0)
    def _():
        txt_bf[...] = txt_ref[...].astype(jnp.bfloat16)

    scale = scale_ref[0]
    bias = bias_ref[0]
    img_blk = (img_ref[...] * scale).astype(jnp.bfloat16)

    n = txt_ref.shape[0]
    total = 0.0
    for j in range(n // _BN):
        x = jax.lax.dot_general(
            img_blk, txt_bf[pl.ds(j * _BN, _BN), :],
            (((1,), (1,)), ((), ())),
            preferred_element_type=jnp.float32,
        )
        x = x + bias
        t = jnp.abs(x)
        u = jnp.exp(-t)
        q = (_Q2 * u + _Q1) * u + _Q0
        body = jnp.maximum(x, 0.0) + u * q
        total = total + jnp.sum(body)
    txt_slice = txt_ref[pl.ds(i * _BM, _BM), :]
    trace = scale * jnp.sum(img_ref[...] * txt_slice) + _BM * bias
    out_ref[i] = total - trace


def kernel(image_features, text_features, logit_scale, logit_bias):
    n, d = image_features.shape
    scale = jnp.reshape(logit_scale.astype(jnp.float32), (1,))
    bias = jnp.reshape(logit_bias.astype(jnp.float32), (1,))
    grid = n // _BM
    partials = pl.pallas_call(
        _stripe_kernel,
        grid=(grid,),
        in_specs=[
            pl.BlockSpec(memory_space=pltpu.SMEM),
            pl.BlockSpec(memory_space=pltpu.SMEM),
            pl.BlockSpec((_BM, d), lambda i: (i, 0)),
            pl.BlockSpec(text_features.shape, lambda i: (0, 0)),
        ],
        out_specs=pl.BlockSpec(memory_space=pltpu.SMEM),
        out_shape=jax.ShapeDtypeStruct((grid,), jnp.float32),
        scratch_shapes=[pltpu.VMEM((n, d), jnp.bfloat16)],
        compiler_params=pltpu.CompilerParams(
            dimension_semantics=("arbitrary",),
        ),
    )(scale, bias, image_features, text_features)
    return jnp.sum(partials) / n
